# baseline (speedup 1.0000x reference)
import jax
import jax.numpy as jnp
from jax import lax
from jax.experimental import pallas as pl
from jax.experimental.pallas import tpu as pltpu

_CONTEXT = 200
_EMBED = 128
_HIDDEN = 128
_VOCAB = 100000

_KB = 25600
_NK = (_CONTEXT * _EMBED) // _KB
_RPC = _KB // _EMBED
_VB = 16384
_NV = -(-_VOCAB // _VB)


def _mlp_body(idx_ref, emb_hbm, w1_ref, b1_ref, w2t_ref, b2_ref, out_ref,
              e_ref, h_ref, sems):
    i = pl.program_id(0)

    def _row_copy(r, chunk):
        return pltpu.make_async_copy(
            emb_hbm.at[pl.ds(idx_ref[r], 1), :],
            e_ref.at[:, pl.ds(r * _EMBED, _EMBED)],
            sems.at[chunk])

    @pl.when(i == 0)
    def _():
        h_ref[...] = jnp.zeros_like(h_ref)

        def issue(r8, c):
            for u in range(8):
                r = r8 * 8 + u
                _row_copy(r, r // _RPC).start()
            return c
        lax.fori_loop(0, _CONTEXT // 8, issue, 0)

    @pl.when(i < _NK)
    def _():
        def drain(r, c):
            _row_copy(r, i).wait()
            return c
        lax.fori_loop(i * _RPC, (i + 1) * _RPC, drain, 0)
        h_ref[...] += jnp.dot(
            e_ref[:, pl.ds(i * _KB, _KB)].astype(jnp.bfloat16),
            w1_ref[...].astype(jnp.bfloat16),
            preferred_element_type=jnp.float32)

    @pl.when(i >= _NK)
    def _():
        h = jnp.maximum(h_ref[...] + b1_ref[...], 0.0)
        out_ref[...] = lax.dot_general(
            h.astype(jnp.bfloat16), w2t_ref[...].astype(jnp.bfloat16),
            (((1,), (1,)), ((), ())),
            preferred_element_type=jnp.float32) + b2_ref[...].reshape(1, _VB)


def kernel(inputs, emb, W1, b1, W2, b2):
    grid = (_NK + _NV,)
    out = pl.pallas_call(
        _mlp_body,
        grid_spec=pltpu.PrefetchScalarGridSpec(
            num_scalar_prefetch=1,
            grid=grid,
            in_specs=[
                pl.BlockSpec(memory_space=pltpu.MemorySpace.HBM),
                pl.BlockSpec((_KB, _HIDDEN),
                             lambda i, idx: (jnp.minimum(i, _NK - 1), 0)),
                pl.BlockSpec((1, _HIDDEN), lambda i, idx: (0, 0)),
                pl.BlockSpec((_VB, _HIDDEN),
                             lambda i, idx: (jnp.maximum(i - _NK, 0), 0)),
                pl.BlockSpec((_VB,),
                             lambda i, idx: (jnp.maximum(i - _NK, 0),)),
            ],
            out_specs=pl.BlockSpec((1, _VB),
                                   lambda i, idx: (0, jnp.maximum(i - _NK, 0))),
            scratch_shapes=[
                pltpu.VMEM((1, _CONTEXT * _EMBED), jnp.float32),
                pltpu.VMEM((1, _HIDDEN), jnp.float32),
                pltpu.SemaphoreType.DMA((_NK,)),
            ],
        ),
        out_shape=jax.ShapeDtypeStruct((1, _VOCAB), jnp.float32),
    )(inputs.astype(jnp.int32), emb, W1, b1.reshape(1, _HIDDEN), W2.T, b2)
    return out


# speedup vs baseline: 1.0572x; 1.0572x over previous
import jax
import jax.numpy as jnp
from jax import lax
from jax.experimental import pallas as pl
from jax.experimental.pallas import tpu as pltpu

_CONTEXT = 200
_EMBED = 128
_HIDDEN = 128
_VOCAB = 100000

_KB = 12800
_NK = (_CONTEXT * _EMBED) // _KB
_RPC = _KB // _EMBED
_VB = 16384
_NV = -(-_VOCAB // _VB)


def _mlp_body(idx_ref, emb_hbm, w1_ref, b1_ref, w2t_ref, b2_ref, out_ref,
              e_ref, h_ref, sems):
    i = pl.program_id(0)

    def _row_copy(r, chunk):
        return pltpu.make_async_copy(
            emb_hbm.at[pl.ds(idx_ref[r], 1), :],
            e_ref.at[:, pl.ds(r * _EMBED, _EMBED)],
            sems.at[chunk])

    @pl.when(i == 0)
    def _():
        h_ref[...] = jnp.zeros_like(h_ref)

        def issue(r8, c):
            for u in range(8):
                r = r8 * 8 + u
                _row_copy(r, r // _RPC).start()
            return c
        lax.fori_loop(0, _CONTEXT // 8, issue, 0)

    @pl.when(i < _NK)
    def _():
        def drain(r, c):
            _row_copy(r, i).wait()
            return c
        lax.fori_loop(i * _RPC, (i + 1) * _RPC, drain, 0)
        h_ref[...] += jnp.dot(
            e_ref[:, pl.ds(i * _KB, _KB)].astype(jnp.bfloat16),
            w1_ref[...].astype(jnp.bfloat16),
            preferred_element_type=jnp.float32)

    @pl.when(i >= _NK)
    def _():
        h = jnp.maximum(h_ref[...] + b1_ref[...], 0.0)
        out_ref[...] = lax.dot_general(
            h.astype(jnp.bfloat16), w2t_ref[...].astype(jnp.bfloat16),
            (((1,), (1,)), ((), ())),
            preferred_element_type=jnp.float32) + b2_ref[...].reshape(1, _VB)


def kernel(inputs, emb, W1, b1, W2, b2):
    grid = (_NK + _NV,)
    out = pl.pallas_call(
        _mlp_body,
        grid_spec=pltpu.PrefetchScalarGridSpec(
            num_scalar_prefetch=1,
            grid=grid,
            in_specs=[
                pl.BlockSpec(memory_space=pltpu.MemorySpace.HBM),
                pl.BlockSpec((_KB, _HIDDEN),
                             lambda i, idx: (jnp.minimum(i, _NK - 1), 0)),
                pl.BlockSpec((1, _HIDDEN), lambda i, idx: (0, 0)),
                pl.BlockSpec((_VB, _HIDDEN),
                             lambda i, idx: (jnp.maximum(i - _NK, 0), 0)),
                pl.BlockSpec((_VB,),
                             lambda i, idx: (jnp.maximum(i - _NK, 0),)),
            ],
            out_specs=pl.BlockSpec((1, _VB),
                                   lambda i, idx: (0, jnp.maximum(i - _NK, 0))),
            scratch_shapes=[
                pltpu.VMEM((1, _CONTEXT * _EMBED), jnp.float32),
                pltpu.VMEM((1, _HIDDEN), jnp.float32),
                pltpu.SemaphoreType.DMA((_NK,)),
            ],
        ),
        out_shape=jax.ShapeDtypeStruct((1, _VOCAB), jnp.float32),
    )(inputs.astype(jnp.int32), emb, W1, b1.reshape(1, _HIDDEN), W2.T, b2)
    return out
